# baseline (speedup 1.0000x reference)
import jax
import jax.numpy as jnp
from jax import lax
from jax.experimental import pallas as pl
from jax.experimental.pallas import tpu as pltpu
from jax.experimental.pallas import tpu_sc as plsc

NUM_USERS = 1000000
NUM_ITEMS = 1000000
EMBED_DIM = 32
BATCH = 16384
TAU = 0.1

NC = 2
NS = 16
L = 16
NW = NC * NS
B_W = BATCH // NW
CHUNK = 128
NCHUNK = B_W // CHUNK
GROUPS = CHUNK // L


def _rsqrt(x):
    xi = plsc.bitcast(x, jnp.int32)
    yi = jnp.int32(0x5F3759DF) - (xi >> 1)
    y = plsc.bitcast(yi, jnp.float32)
    half = jnp.float32(0.5) * x
    for _ in range(3):
        y = y * (jnp.float32(1.5) - half * y * y)
    return y


def _sc_body(users_hbm, items_hbm, utT, itT, out_hbm,
             idx_u, idx_i, u_rows, i_rows, out_buf, sem_u, sem_i):
    wid = lax.axis_index("s") * NC + lax.axis_index("c")
    base = wid * B_W

    pltpu.sync_copy(users_hbm.at[pl.ds(base, B_W)], idx_u)
    pltpu.sync_copy(items_hbm.at[pl.ds(base, B_W)], idx_i)

    def issue(c, buf):
        sl = pl.ds(c * CHUNK, CHUNK)
        for d in range(EMBED_DIM):
            pltpu.async_copy(utT.at[d].at[idx_u.at[sl]], u_rows.at[buf, d], sem_u)
            pltpu.async_copy(itT.at[d].at[idx_i.at[sl]], i_rows.at[buf, d], sem_i)

    def drain(buf):
        for d in range(EMBED_DIM):
            pltpu.make_async_copy(utT.at[0].at[idx_u.at[pl.ds(0, CHUNK)]],
                                  u_rows.at[buf, d], sem_u).wait()
            pltpu.make_async_copy(itT.at[0].at[idx_i.at[pl.ds(0, CHUNK)]],
                                  i_rows.at[buf, d], sem_i).wait()

    def compute(c, buf):
        def group(g, _):
            sl = pl.ds(g * L, L)
            acc_dot = jnp.zeros((L,), jnp.float32)
            acc_nu = jnp.zeros((L,), jnp.float32)
            acc_ni = jnp.zeros((L,), jnp.float32)
            for d in range(EMBED_DIM):
                u = u_rows[buf, d, sl]
                v = i_rows[buf, d, sl]
                acc_dot = acc_dot + u * v
                acc_nu = acc_nu + u * u
                acc_ni = acc_ni + v * v
            nu = jnp.maximum(acc_nu, jnp.float32(1e-24))
            ni = jnp.maximum(acc_ni, jnp.float32(1e-24))
            res = acc_dot * _rsqrt(nu) * _rsqrt(ni) / jnp.float32(TAU)
            out_buf[pl.ds(c * CHUNK + g * L, L)] = res
            return 0

        lax.fori_loop(0, GROUPS, group, 0)

    issue(0, 0)
    for c in range(NCHUNK):
        if c + 1 < NCHUNK:
            issue(c + 1, (c + 1) % 2)
        drain(c % 2)
        compute(c, c % 2)

    pltpu.sync_copy(out_buf, out_hbm.at[pl.ds(base, B_W)])


@jax.jit
def kernel(users, items, user_embedding, item_embedding):
    mesh = plsc.VectorSubcoreMesh(core_axis_name="c", subcore_axis_name="s",
                                  num_cores=NC, num_subcores=NS)
    run = pl.kernel(
        _sc_body,
        out_type=jax.ShapeDtypeStruct((BATCH,), jnp.float32),
        mesh=mesh,
        compiler_params=pltpu.CompilerParams(needs_layout_passes=False,
                                             use_tc_tiling_on_sc=False),
        scratch_types=[
            pltpu.VMEM((B_W,), jnp.int32),
            pltpu.VMEM((B_W,), jnp.int32),
            pltpu.VMEM((2, EMBED_DIM, CHUNK), jnp.float32),
            pltpu.VMEM((2, EMBED_DIM, CHUNK), jnp.float32),
            pltpu.VMEM((B_W,), jnp.float32),
            pltpu.SemaphoreType.DMA,
            pltpu.SemaphoreType.DMA,
        ],
    )
    return run(users, items, user_embedding.T, item_embedding.T)


# speedup vs baseline: 5.7104x; 5.7104x over previous
import jax
import jax.numpy as jnp
from jax import lax
from jax.experimental import pallas as pl
from jax.experimental.pallas import tpu as pltpu
from jax.experimental.pallas import tpu_sc as plsc

NUM_USERS = 1000000
NUM_ITEMS = 1000000
EMBED_DIM = 32
BATCH = 16384
TAU = 0.1

NC = 2
NS = 16
L = 16
NW = NC * NS
B_W = BATCH // NW
CHUNK = 128
NCHUNK = B_W // CHUNK
GROUPS = CHUNK // L


def _rsqrt(x):
    xi = plsc.bitcast(x, jnp.int32)
    yi = jnp.int32(0x5F3759DF) - (xi >> 1)
    y = plsc.bitcast(yi, jnp.float32)
    half = jnp.float32(0.5) * x
    for _ in range(3):
        y = y * (jnp.float32(1.5) - half * y * y)
    return y


def _sc_body(users_hbm, items_hbm, uemb_hbm, iemb_hbm, out_hbm,
             idx_u, idx_i, u_rows, i_rows, out_buf, sem_u, sem_i):
    wid = lax.axis_index("s") * NC + lax.axis_index("c")
    base = wid * B_W

    lane = lax.iota(jnp.int32, L)

    pltpu.sync_copy(users_hbm.at[pl.ds(base, B_W)], idx_u)
    pltpu.sync_copy(items_hbm.at[pl.ds(base, B_W)], idx_i)

    def issue(c, buf):
        sl = pl.ds(c * CHUNK, CHUNK)
        pltpu.async_copy(uemb_hbm.at[idx_u.at[sl]], u_rows.at[buf], sem_u)
        pltpu.async_copy(iemb_hbm.at[idx_i.at[sl]], i_rows.at[buf], sem_i)

    def drain(buf):
        pltpu.make_async_copy(uemb_hbm.at[idx_u.at[pl.ds(0, CHUNK)]],
                              u_rows.at[buf], sem_u).wait()
        pltpu.make_async_copy(iemb_hbm.at[idx_i.at[pl.ds(0, CHUNK)]],
                              i_rows.at[buf], sem_i).wait()

    def compute(c, buf):
        def group(g, _):
            row = g * L + lane
            acc_dot = jnp.zeros((L,), jnp.float32)
            acc_nu = jnp.zeros((L,), jnp.float32)
            acc_ni = jnp.zeros((L,), jnp.float32)
            for d in range(EMBED_DIM):
                col = (lane + d) & (EMBED_DIM - 1)
                u = plsc.load_gather(u_rows, [jnp.zeros((L,), jnp.int32) + buf,
                                              row, col])
                v = plsc.load_gather(i_rows, [jnp.zeros((L,), jnp.int32) + buf,
                                              row, col])
                acc_dot = acc_dot + u * v
                acc_nu = acc_nu + u * u
                acc_ni = acc_ni + v * v
            nu = jnp.maximum(acc_nu, jnp.float32(1e-24))
            ni = jnp.maximum(acc_ni, jnp.float32(1e-24))
            res = acc_dot * _rsqrt(nu) * _rsqrt(ni) / jnp.float32(TAU)
            out_buf[pl.ds(c * CHUNK + g * L, L)] = res
            return 0

        lax.fori_loop(0, GROUPS, group, 0)

    issue(0, 0)
    for c in range(NCHUNK):
        if c + 1 < NCHUNK:
            issue(c + 1, (c + 1) % 2)
        drain(c % 2)
        compute(c, c % 2)

    pltpu.sync_copy(out_buf, out_hbm.at[pl.ds(base, B_W)])


@jax.jit
def kernel(users, items, user_embedding, item_embedding):
    mesh = plsc.VectorSubcoreMesh(core_axis_name="c", subcore_axis_name="s",
                                  num_cores=NC, num_subcores=NS)
    run = pl.kernel(
        _sc_body,
        out_type=jax.ShapeDtypeStruct((BATCH,), jnp.float32),
        mesh=mesh,
        compiler_params=pltpu.CompilerParams(needs_layout_passes=False,
                                             use_tc_tiling_on_sc=False),
        scratch_types=[
            pltpu.VMEM((B_W,), jnp.int32),
            pltpu.VMEM((B_W,), jnp.int32),
            pltpu.VMEM((2, CHUNK, EMBED_DIM), jnp.float32),
            pltpu.VMEM((2, CHUNK, EMBED_DIM), jnp.float32),
            pltpu.VMEM((B_W,), jnp.float32),
            pltpu.SemaphoreType.DMA,
            pltpu.SemaphoreType.DMA,
        ],
    )
    return run(users, items, user_embedding, item_embedding)


# speedup vs baseline: 21.5840x; 3.7797x over previous
import jax
import jax.numpy as jnp
from jax import lax
from jax.experimental import pallas as pl
from jax.experimental.pallas import tpu as pltpu
from jax.experimental.pallas import tpu_sc as plsc

NUM_USERS = 1000000
NUM_ITEMS = 1000000
EMBED_DIM = 32
BATCH = 16384
TAU = 0.1

NC = 2
NS = 16
L = 16
NW = NC * NS
B_W = BATCH // NW
PE = 4
NPH = B_W // PE
GROUPS = B_W // L
TROWS = EMBED_DIM // 8


def _rsqrt(x):
    xi = plsc.bitcast(x, jnp.int32)
    yi = jnp.int32(0x5F3759DF) - (xi >> 1)
    y = plsc.bitcast(yi, jnp.float32)
    half = jnp.float32(0.5) * x
    for _ in range(3):
        y = y * (jnp.float32(1.5) - half * y * y)
    return y


def _sc_body(users_hbm, items_hbm, utT, itT, out_hbm,
             vidx_u, vidx_i, tiles, u_rows, i_rows, out_buf,
             sem):
    wid = lax.axis_index("s") * NC + lax.axis_index("c")
    base = wid * B_W

    lane = lax.iota(jnp.int32, L)

    pltpu.sync_copy(users_hbm.at[pl.ds(base, B_W)], vidx_u.at[pl.ds(0, B_W)])
    pltpu.sync_copy(items_hbm.at[pl.ds(base, B_W)], vidx_i.at[pl.ds(0, B_W)])

    def pass_one(tbl, vidx, rows):
        def issue(p, buf):
            vec = vidx[pl.ds(p * PE, L)]
            for j in range(PE):
                col = pl.multiple_of((vec[j] >> 7) * 128, 128)
                for r in range(TROWS):
                    pltpu.async_copy(tbl.at[pl.ds(r * 8, 8), pl.ds(col, 128)],
                                     tiles.at[buf * (PE * TROWS) + j * TROWS + r],
                                     sem)

        def drain(buf):
            for j in range(PE):
                for r in range(TROWS):
                    pltpu.make_async_copy(tbl.at[pl.ds(0, 8), pl.ds(0, 128)],
                                          tiles.at[buf * (PE * TROWS) + j * TROWS + r],
                                          sem).wait()

        def extract(p, buf):
            el = lane & (PE - 1)
            mask = lane < PE
            u = plsc.load_gather(vidx, [p * PE + el])
            ulane = u & 127
            bufv = jnp.zeros((L,), jnp.int32) + buf
            for d in range(EMBED_DIM):
                dd = (d + 0) & (EMBED_DIM - 1)
                rv = jnp.zeros((L,), jnp.int32) + (dd >> 3)
                sv = jnp.zeros((L,), jnp.int32) + (dd & 7)
                val = plsc.load_gather(
                    tiles, [bufv * (PE * TROWS) + el * TROWS + rv, sv, ulane])
                plsc.store_scatter(rows, [jnp.zeros((L,), jnp.int32) + dd,
                                          p * PE + el],
                                   val, mask=mask)

        issue(0, 0)

        def phase(p, _):
            @pl.when(p + 1 < NPH)
            def _():
                issue(p + 1, (p + 1) % 2)

            drain(p % 2)
            extract(p, p % 2)
            return 0

        lax.fori_loop(0, NPH, phase, 0)

    pass_one(utT, vidx_u, u_rows)
    pass_one(itT, vidx_i, i_rows)

    def group(g, _):
        row = g * L + lane
        acc_dot = jnp.zeros((L,), jnp.float32)
        acc_nu = jnp.zeros((L,), jnp.float32)
        acc_ni = jnp.zeros((L,), jnp.float32)
        for d in range(EMBED_DIM):
            col = (lane + d) & (EMBED_DIM - 1)
            u = plsc.load_gather(u_rows, [col, row])
            v = plsc.load_gather(i_rows, [col, row])
            acc_dot = acc_dot + u * v
            acc_nu = acc_nu + u * u
            acc_ni = acc_ni + v * v
        nu = jnp.maximum(acc_nu, jnp.float32(1e-24))
        ni = jnp.maximum(acc_ni, jnp.float32(1e-24))
        res = acc_dot * _rsqrt(nu) * _rsqrt(ni) / jnp.float32(TAU)
        out_buf[pl.ds(g * L, L)] = res
        return 0

    lax.fori_loop(0, GROUPS, group, 0)

    pltpu.sync_copy(out_buf, out_hbm.at[pl.ds(base, B_W)])


@jax.jit
def kernel(users, items, user_embedding, item_embedding):
    mesh = plsc.VectorSubcoreMesh(core_axis_name="c", subcore_axis_name="s",
                                  num_cores=NC, num_subcores=NS)
    run = pl.kernel(
        _sc_body,
        out_type=jax.ShapeDtypeStruct((BATCH,), jnp.float32),
        mesh=mesh,
        compiler_params=pltpu.CompilerParams(needs_layout_passes=False,
                                             use_tc_tiling_on_sc=True),
        scratch_types=[
            pltpu.VMEM((B_W + L,), jnp.int32),
            pltpu.VMEM((B_W + L,), jnp.int32),
            pltpu.VMEM((2 * PE * TROWS, 8, 128), jnp.float32),
            pltpu.VMEM((EMBED_DIM, B_W), jnp.float32),
            pltpu.VMEM((EMBED_DIM, B_W), jnp.float32),
            pltpu.VMEM((B_W,), jnp.float32),
            pltpu.SemaphoreType.DMA,
        ],
    )
    return run(users, items, user_embedding.T, item_embedding.T)


# speedup vs baseline: 25.2576x; 1.1702x over previous
import jax
import jax.numpy as jnp
from jax import lax
from jax.experimental import pallas as pl
from jax.experimental.pallas import tpu as pltpu
from jax.experimental.pallas import tpu_sc as plsc

NUM_USERS = 1000000
NUM_ITEMS = 1000000
EMBED_DIM = 32
BATCH = 16384
TAU = 0.1

NC = 2
NS = 16
L = 16
NW = NC * NS
B_W = BATCH // NW
PE = 8
NPH = B_W // PE
GROUPS = B_W // L
TROWS = EMBED_DIM // 8


def _rsqrt(x):
    xi = plsc.bitcast(x, jnp.int32)
    yi = jnp.int32(0x5F3759DF) - (xi >> 1)
    y = plsc.bitcast(yi, jnp.float32)
    half = jnp.float32(0.5) * x
    for _ in range(3):
        y = y * (jnp.float32(1.5) - half * y * y)
    return y


def _sc_body(users_hbm, items_hbm, utT, itT, out_hbm,
             vidx_u, vidx_i, tiles, u_rows, i_rows, out_buf,
             sem):
    wid = lax.axis_index("s") * NC + lax.axis_index("c")
    base = wid * B_W

    lane = lax.iota(jnp.int32, L)

    pltpu.sync_copy(users_hbm.at[pl.ds(base, B_W)], vidx_u.at[pl.ds(0, B_W)])
    pltpu.sync_copy(items_hbm.at[pl.ds(base, B_W)], vidx_i.at[pl.ds(0, B_W)])

    def pass_one(tbl, vidx, rows):
        def issue(p, buf):
            vec = vidx[pl.ds(p * PE, L)]
            for j in range(PE):
                col = pl.multiple_of((vec[j] >> 7) * 128, 128)
                for r in range(TROWS):
                    pltpu.async_copy(tbl.at[pl.ds(r * 8, 8), pl.ds(col, 128)],
                                     tiles.at[buf * (PE * TROWS) + j * TROWS + r],
                                     sem)

        def drain(buf):
            for j in range(PE):
                for r in range(TROWS):
                    pltpu.make_async_copy(tbl.at[pl.ds(0, 8), pl.ds(0, 128)],
                                          tiles.at[buf * (PE * TROWS) + j * TROWS + r],
                                          sem).wait()

        def extract(p, buf):
            el = lane & (PE - 1)
            mask = lane < PE
            u = plsc.load_gather(vidx, [p * PE + el])
            ulane = u & 127
            bufv = jnp.zeros((L,), jnp.int32) + buf
            for d in range(EMBED_DIM):
                dd = (d + 0) & (EMBED_DIM - 1)
                rv = jnp.zeros((L,), jnp.int32) + (dd >> 3)
                sv = jnp.zeros((L,), jnp.int32) + (dd & 7)
                val = plsc.load_gather(
                    tiles, [bufv * (PE * TROWS) + el * TROWS + rv, sv, ulane])
                plsc.store_scatter(rows, [jnp.zeros((L,), jnp.int32) + dd,
                                          p * PE + el],
                                   val, mask=mask)

        issue(0, 0)

        def phase(p, _):
            @pl.when(p + 1 < NPH)
            def _():
                issue(p + 1, (p + 1) % 2)

            drain(p % 2)
            extract(p, p % 2)
            return 0

        lax.fori_loop(0, NPH, phase, 0)

    pass_one(utT, vidx_u, u_rows)
    pass_one(itT, vidx_i, i_rows)

    def group(g, _):
        row = g * L + lane
        acc_dot = jnp.zeros((L,), jnp.float32)
        acc_nu = jnp.zeros((L,), jnp.float32)
        acc_ni = jnp.zeros((L,), jnp.float32)
        for d in range(EMBED_DIM):
            col = (lane + d) & (EMBED_DIM - 1)
            u = plsc.load_gather(u_rows, [col, row])
            v = plsc.load_gather(i_rows, [col, row])
            acc_dot = acc_dot + u * v
            acc_nu = acc_nu + u * u
            acc_ni = acc_ni + v * v
        nu = jnp.maximum(acc_nu, jnp.float32(1e-24))
        ni = jnp.maximum(acc_ni, jnp.float32(1e-24))
        res = acc_dot * _rsqrt(nu) * _rsqrt(ni) / jnp.float32(TAU)
        out_buf[pl.ds(g * L, L)] = res
        return 0

    lax.fori_loop(0, GROUPS, group, 0)

    pltpu.sync_copy(out_buf, out_hbm.at[pl.ds(base, B_W)])


@jax.jit
def kernel(users, items, user_embedding, item_embedding):
    mesh = plsc.VectorSubcoreMesh(core_axis_name="c", subcore_axis_name="s",
                                  num_cores=NC, num_subcores=NS)
    run = pl.kernel(
        _sc_body,
        out_type=jax.ShapeDtypeStruct((BATCH,), jnp.float32),
        mesh=mesh,
        compiler_params=pltpu.CompilerParams(needs_layout_passes=False,
                                             use_tc_tiling_on_sc=True),
        scratch_types=[
            pltpu.VMEM((B_W + L,), jnp.int32),
            pltpu.VMEM((B_W + L,), jnp.int32),
            pltpu.VMEM((2 * PE * TROWS, 8, 128), jnp.float32),
            pltpu.VMEM((EMBED_DIM, B_W), jnp.float32),
            pltpu.VMEM((EMBED_DIM, B_W), jnp.float32),
            pltpu.VMEM((B_W,), jnp.float32),
            pltpu.SemaphoreType.DMA,
        ],
    )
    return run(users, items, user_embedding.T, item_embedding.T)
